# Initial kernel scaffold; baseline (speedup 1.0000x reference)
import functools

import jax
import jax.numpy as jnp
from jax import lax
from jax.experimental import pallas as pl
from jax.experimental.pallas import tpu as pltpu
from jax.experimental.pallas import tpu_sc as plsc

I_T = 10000
J_W = 1000
K_C = 32
N_L = 10000

NC = 2
NS = 16
NW = NC * NS

P_LBL = 320
N_PAD = NW * P_LBL
CH = 64
J_PAD = 1008
IP = 10240
SCAT_SL = IP * K_C // NS
BASE_SL = IP // NS


def _ad_body(s_ref, a_ref, d_ref):
    s = s_ref[...]
    sig = jax.nn.sigmoid(s)
    noi = jax.nn.sigmoid(-s)
    tot = sig + noi
    a = jnp.log(noi / (K_C * tot))
    b = jnp.log((sig + noi / K_C) / tot)
    a_ref[...] = a
    d_ref[...] = b - a


def _seg_body(ii_hbm, jj_hbm, y_hbm, a_hbm, d_hbm, zer_hbm,
              scat_out, base_out,
              ii_v, jj_v, y_v, a_v, d_v, idx_b, val_b, iib_b, av_b,
              scat_sh, base_sh):
    c = lax.axis_index("c")
    s = lax.axis_index("s")
    wid = s * NC + c
    lbl0 = wid * P_LBL

    pltpu.sync_copy(ii_hbm.at[pl.ds(lbl0, P_LBL)], ii_v)
    pltpu.sync_copy(jj_hbm.at[pl.ds(lbl0, P_LBL)], jj_v)
    pltpu.sync_copy(y_hbm.at[pl.ds(lbl0, P_LBL)], y_v)
    pltpu.sync_copy(a_hbm, a_v)
    pltpu.sync_copy(d_hbm, d_v)

    pltpu.sync_copy(zer_hbm, scat_sh.at[pl.ds(s * SCAT_SL, SCAT_SL)])
    pltpu.sync_copy(zer_hbm.at[pl.ds(0, BASE_SL)],
                    base_sh.at[pl.ds(s * BASE_SL, BASE_SL)])
    plsc.subcore_barrier()

    for chunk in range(P_LBL // CH):
        for v in range(CH // 16):
            off = chunk * CH + v * 16
            iiv = ii_v[pl.ds(off, 16)]
            jjv = jj_v[pl.ds(off, 16)]
            yv = y_v[pl.ds(off, 16)]
            av = plsc.load_gather(a_v, [jjv])
            dv = plsc.load_gather(d_v, [jjv])
            idx_b[pl.ds(v * 16, 16)] = iiv * K_C + yv
            val_b[pl.ds(v * 16, 16)] = dv
            iib_b[pl.ds(v * 16, 16)] = iiv
            av_b[pl.ds(v * 16, 16)] = av
        pltpu.sync_copy(val_b, scat_sh.at[idx_b], add=True)
        pltpu.sync_copy(av_b, base_sh.at[iib_b], add=True)

    plsc.subcore_barrier()
    pltpu.sync_copy(scat_sh.at[pl.ds(s * SCAT_SL, SCAT_SL)], scat_out.at[c, s])
    pltpu.sync_copy(base_sh.at[pl.ds(s * BASE_SL, BASE_SL)], base_out.at[c, s])


_seg_kernel = functools.partial(
    pl.kernel,
    mesh=plsc.VectorSubcoreMesh(core_axis_name="c", subcore_axis_name="s"),
    out_type=[
        jax.ShapeDtypeStruct((NC, NS, SCAT_SL), jnp.float32),
        jax.ShapeDtypeStruct((NC, NS, BASE_SL), jnp.float32),
    ],
    scratch_types=[
        pltpu.VMEM((P_LBL,), jnp.int32),
        pltpu.VMEM((P_LBL,), jnp.int32),
        pltpu.VMEM((P_LBL,), jnp.int32),
        pltpu.VMEM((J_PAD,), jnp.float32),
        pltpu.VMEM((J_PAD,), jnp.float32),
        pltpu.VMEM((CH,), jnp.int32),
        pltpu.VMEM((CH,), jnp.float32),
        pltpu.VMEM((CH,), jnp.int32),
        pltpu.VMEM((CH,), jnp.float32),
        pltpu.VMEM_SHARED((IP * K_C,), jnp.float32),
        pltpu.VMEM_SHARED((IP,), jnp.float32),
    ],
)(_seg_body)


def _post_body(scat_ref, base_ref, qz_ref, vq_ref):
    scat = jnp.sum(scat_ref[...], axis=0)
    basec = jnp.sum(base_ref[...], axis=0)
    m = jnp.max(scat, axis=-1, keepdims=True)
    e = jnp.exp(scat - m)
    z = jnp.sum(e, axis=-1, keepdims=True)
    qz_ref[...] = e / z
    vq_ref[...] = basec + m + jnp.log(z)


def kernel(ii, jj, y, snr_logit):
    ii = ii.astype(jnp.int32)
    jj = jj.astype(jnp.int32)
    y = y.astype(jnp.int32)
    f32 = jnp.float32

    a8, d8 = pl.pallas_call(
        _ad_body,
        out_shape=[jax.ShapeDtypeStruct((8, J_W // 8), f32)] * 2,
    )(snr_logit.reshape(8, J_W // 8))
    zpad = jnp.zeros((J_PAD - J_W,), f32)
    a_p = jnp.concatenate([a8.reshape(-1), zpad])
    d_p = jnp.concatenate([d8.reshape(-1), zpad])

    pad = N_PAD - N_L
    ii_p = jnp.concatenate([ii, jnp.zeros((pad,), jnp.int32)])
    jj_p = jnp.concatenate([jj, jnp.full((pad,), J_W, jnp.int32)])
    y_p = jnp.concatenate([y, jnp.zeros((pad,), jnp.int32)])
    zer = jnp.zeros((SCAT_SL,), f32)

    scat_part, base_part = _seg_kernel(ii_p, jj_p, y_p, a_p, d_p, zer)
    scat2 = scat_part.reshape(NC, IP, K_C)[:, :I_T]
    base2 = base_part.reshape(NC, IP, 1)[:, :I_T]

    R = 1000
    grid = (I_T // R,)
    qz, vq = pl.pallas_call(
        _post_body,
        grid=grid,
        in_specs=[
            pl.BlockSpec((NC, R, K_C), lambda i: (0, i, 0)),
            pl.BlockSpec((NC, R, 1), lambda i: (0, i, 0)),
        ],
        out_specs=[
            pl.BlockSpec((R, K_C), lambda i: (i, 0)),
            pl.BlockSpec((R, 1), lambda i: (i, 0)),
        ],
        out_shape=[
            jax.ShapeDtypeStruct((I_T, K_C), f32),
            jax.ShapeDtypeStruct((I_T, 1), f32),
        ],
    )(scat2, base2)
    return qz, vq.reshape(-1)


# speedup vs baseline: 1.5965x; 1.5965x over previous
import functools

import jax
import jax.numpy as jnp
from jax import lax
from jax.experimental import pallas as pl
from jax.experimental.pallas import tpu as pltpu
from jax.experimental.pallas import tpu_sc as plsc

I_T = 10000
J_W = 1000
K_C = 32
N_L = 10000

NC = 2
NS = 16
NW = NC * NS

P_LBL = 320
N_PAD = NW * P_LBL
CH = 64
J_PAD = 1008
IP = 10240
SCAT_SL = IP * K_C // NS
BASE_SL = IP // NS


def _ad_body(s_ref, a_ref, d_ref):
    s = s_ref[...]
    sig = jax.nn.sigmoid(s)
    noi = jax.nn.sigmoid(-s)
    tot = sig + noi
    a = jnp.log(noi / (K_C * tot))
    b = jnp.log((sig + noi / K_C) / tot)
    a_ref[...] = a
    d_ref[...] = b - a


def _seg_body(ii_hbm, jj_hbm, y_hbm, a_hbm, d_hbm, zer_hbm,
              scat_out, base_out,
              ii_v, jj_v, y_v, a_v, d_v, idx_b, val_b, iib_b, av_b,
              scat_sh, base_sh):
    c = lax.axis_index("c")
    s = lax.axis_index("s")
    wid = s * NC + c
    lbl0 = wid * P_LBL

    pltpu.sync_copy(ii_hbm.at[pl.ds(lbl0, P_LBL)], ii_v)
    pltpu.sync_copy(jj_hbm.at[pl.ds(lbl0, P_LBL)], jj_v)
    pltpu.sync_copy(y_hbm.at[pl.ds(lbl0, P_LBL)], y_v)
    pltpu.sync_copy(a_hbm, a_v)
    pltpu.sync_copy(d_hbm, d_v)

    pltpu.sync_copy(zer_hbm, scat_sh.at[pl.ds(s * SCAT_SL, SCAT_SL)])
    pltpu.sync_copy(zer_hbm.at[pl.ds(0, BASE_SL)],
                    base_sh.at[pl.ds(s * BASE_SL, BASE_SL)])
    plsc.subcore_barrier()

    for chunk in range(P_LBL // CH):
        for v in range(CH // 16):
            off = chunk * CH + v * 16
            iiv = ii_v[pl.ds(off, 16)]
            jjv = jj_v[pl.ds(off, 16)]
            yv = y_v[pl.ds(off, 16)]
            av = plsc.load_gather(a_v, [jjv])
            dv = plsc.load_gather(d_v, [jjv])
            idx_b[pl.ds(v * 16, 16)] = iiv * K_C + yv
            val_b[pl.ds(v * 16, 16)] = dv
            iib_b[pl.ds(v * 16, 16)] = iiv
            av_b[pl.ds(v * 16, 16)] = av
        pltpu.sync_copy(val_b, scat_sh.at[idx_b], add=True)
        pltpu.sync_copy(av_b, base_sh.at[iib_b], add=True)

    plsc.subcore_barrier()
    pltpu.sync_copy(scat_sh.at[pl.ds(s * SCAT_SL, SCAT_SL)], scat_out.at[c, s])
    pltpu.sync_copy(base_sh.at[pl.ds(s * BASE_SL, BASE_SL)], base_out.at[c, s])


_seg_kernel = functools.partial(
    pl.kernel,
    mesh=plsc.VectorSubcoreMesh(core_axis_name="c", subcore_axis_name="s"),
    compiler_params=pltpu.CompilerParams(needs_layout_passes=False),
    out_type=[
        jax.ShapeDtypeStruct((NC, NS, SCAT_SL), jnp.float32),
        jax.ShapeDtypeStruct((NC, NS, BASE_SL), jnp.float32),
    ],
    scratch_types=[
        pltpu.VMEM((P_LBL,), jnp.int32),
        pltpu.VMEM((P_LBL,), jnp.int32),
        pltpu.VMEM((P_LBL,), jnp.int32),
        pltpu.VMEM((J_PAD,), jnp.float32),
        pltpu.VMEM((J_PAD,), jnp.float32),
        pltpu.VMEM((CH,), jnp.int32),
        pltpu.VMEM((CH,), jnp.float32),
        pltpu.VMEM((CH,), jnp.int32),
        pltpu.VMEM((CH,), jnp.float32),
        pltpu.VMEM_SHARED((IP * K_C,), jnp.float32),
        pltpu.VMEM_SHARED((IP,), jnp.float32),
    ],
)(_seg_body)


def _post_body(scat_ref, base_ref, qz_ref, vq_ref):
    scat = jnp.sum(scat_ref[...], axis=0)
    basec = jnp.sum(base_ref[...], axis=0)
    m = jnp.max(scat, axis=-1, keepdims=True)
    e = jnp.exp(scat - m)
    z = jnp.sum(e, axis=-1, keepdims=True)
    qz_ref[...] = e / z
    vq_ref[...] = basec + m + jnp.log(z)


def kernel(ii, jj, y, snr_logit):
    ii = ii.astype(jnp.int32)
    jj = jj.astype(jnp.int32)
    y = y.astype(jnp.int32)
    f32 = jnp.float32

    a8, d8 = pl.pallas_call(
        _ad_body,
        out_shape=[jax.ShapeDtypeStruct((8, J_W // 8), f32)] * 2,
    )(snr_logit.reshape(8, J_W // 8))
    zpad = jnp.zeros((J_PAD - J_W,), f32)
    a_p = jnp.concatenate([a8.reshape(-1), zpad])
    d_p = jnp.concatenate([d8.reshape(-1), zpad])

    pad = N_PAD - N_L
    ii_p = jnp.concatenate([ii, jnp.zeros((pad,), jnp.int32)])
    jj_p = jnp.concatenate([jj, jnp.full((pad,), J_W, jnp.int32)])
    y_p = jnp.concatenate([y, jnp.zeros((pad,), jnp.int32)])
    zer = jnp.zeros((SCAT_SL,), f32)

    scat_part, base_part = _seg_kernel(ii_p, jj_p, y_p, a_p, d_p, zer)
    scat2 = scat_part.reshape(NC, IP, K_C)[:, :I_T]
    base2 = base_part.reshape(NC, IP, 1)[:, :I_T]

    R = 1000
    grid = (I_T // R,)
    qz, vq = pl.pallas_call(
        _post_body,
        grid=grid,
        in_specs=[
            pl.BlockSpec((NC, R, K_C), lambda i: (0, i, 0)),
            pl.BlockSpec((NC, R, 1), lambda i: (0, i, 0)),
        ],
        out_specs=[
            pl.BlockSpec((R, K_C), lambda i: (i, 0)),
            pl.BlockSpec((R, 1), lambda i: (i, 0)),
        ],
        out_shape=[
            jax.ShapeDtypeStruct((I_T, K_C), f32),
            jax.ShapeDtypeStruct((I_T, 1), f32),
        ],
    )(scat2, base2)
    return qz, vq.reshape(-1)


# speedup vs baseline: 1.7000x; 1.0648x over previous
import functools

import jax
import jax.numpy as jnp
from jax import lax
from jax.experimental import pallas as pl
from jax.experimental.pallas import tpu as pltpu
from jax.experimental.pallas import tpu_sc as plsc

I_T = 10000
J_W = 1000
K_C = 32
N_L = 10000

NC = 2
NS = 16
NW = NC * NS

P_LBL = 320
CH = 64
I_PAD = 10240
SCAT_SL = I_PAD * K_C // NS
BASE_SL = I_PAD // NS
ROWS = 320
LN2 = 0.6931471805599453


def _ad_body(s_ref, a_ref, d_ref):
    s = s_ref[...]
    sig = jax.nn.sigmoid(s)
    noi = jax.nn.sigmoid(-s)
    tot = sig + noi
    a = jnp.log(noi / (K_C * tot))
    b = jnp.log((sig + noi / K_C) / tot)
    a_ref[...] = a
    d_ref[...] = b - a


def _seg_body(ii_hbm, jj_hbm, y_hbm, a_hbm, d_hbm, zer_hbm,
              scat_out, base_out,
              ii_v, jj_v, y_v, a_v, d_v, idx_b, val_b, iib_b, av_b,
              scat_sh, base_sh, sem):
    c = lax.axis_index("c")
    s = lax.axis_index("s")
    wid = s * NC + c
    start = wid * P_LBL
    l0 = jnp.minimum(start, N_L - P_LBL)
    l0 = pl.multiple_of(l0, 8)

    cps = [
        pltpu.async_copy(ii_hbm.at[pl.ds(l0, P_LBL)], ii_v, sem),
        pltpu.async_copy(jj_hbm.at[pl.ds(l0, P_LBL)], jj_v, sem),
        pltpu.async_copy(y_hbm.at[pl.ds(l0, P_LBL)], y_v, sem),
        pltpu.async_copy(a_hbm, a_v, sem),
        pltpu.async_copy(d_hbm, d_v, sem),
        pltpu.async_copy(zer_hbm, scat_sh.at[pl.ds(s * SCAT_SL, SCAT_SL)], sem),
        pltpu.async_copy(zer_hbm.at[pl.ds(0, BASE_SL)],
                         base_sh.at[pl.ds(s * BASE_SL, BASE_SL)], sem),
    ]
    for cp in cps:
        cp.wait()
    plsc.subcore_barrier()

    for chunk in range(P_LBL // CH):
        for v in range(CH // 16):
            off = chunk * CH + v * 16
            iiv = ii_v[pl.ds(off, 16)]
            jjv = jj_v[pl.ds(off, 16)]
            yv = y_v[pl.ds(off, 16)]
            av = plsc.load_gather(a_v, [jjv])
            dv = plsc.load_gather(d_v, [jjv])
            g = l0 + off + lax.iota(jnp.int32, 16)
            ok = g >= start
            zero = jnp.zeros((16,), jnp.float32)
            idx_b[pl.ds(v * 16, 16)] = iiv * K_C + yv
            val_b[pl.ds(v * 16, 16)] = jnp.where(ok, dv, zero)
            iib_b[pl.ds(v * 16, 16)] = iiv
            av_b[pl.ds(v * 16, 16)] = jnp.where(ok, av, zero)
        pltpu.sync_copy(val_b, scat_sh.at[idx_b], add=True)
        pltpu.sync_copy(av_b, base_sh.at[iib_b], add=True)

    plsc.subcore_barrier()
    pltpu.sync_copy(scat_sh.at[pl.ds(s * SCAT_SL, SCAT_SL)], scat_out.at[c, s])
    pltpu.sync_copy(base_sh.at[pl.ds(s * BASE_SL, BASE_SL)], base_out.at[c, s])


_seg_kernel = functools.partial(
    pl.kernel,
    mesh=plsc.VectorSubcoreMesh(core_axis_name="c", subcore_axis_name="s"),
    compiler_params=pltpu.CompilerParams(needs_layout_passes=False),
    out_type=[
        jax.ShapeDtypeStruct((NC, NS, SCAT_SL), jnp.float32),
        jax.ShapeDtypeStruct((NC, NS, BASE_SL), jnp.float32),
    ],
    scratch_types=[
        pltpu.VMEM((P_LBL,), jnp.int32),
        pltpu.VMEM((P_LBL,), jnp.int32),
        pltpu.VMEM((P_LBL,), jnp.int32),
        pltpu.VMEM((J_W,), jnp.float32),
        pltpu.VMEM((J_W,), jnp.float32),
        pltpu.VMEM((CH,), jnp.int32),
        pltpu.VMEM((CH,), jnp.float32),
        pltpu.VMEM((CH,), jnp.int32),
        pltpu.VMEM((CH,), jnp.float32),
        pltpu.VMEM_SHARED((I_PAD * K_C,), jnp.float32),
        pltpu.VMEM_SHARED((I_PAD,), jnp.float32),
        pltpu.SemaphoreType.DMA,
    ],
)(_seg_body)


def _log_f32(x):
    bits = plsc.bitcast(x, jnp.int32)
    e = (bits >> 23) - 127
    m = plsc.bitcast((bits & 0x7FFFFF) | 0x3F800000, jnp.float32)
    s = (m - 1.0) / (m + 1.0)
    s2 = s * s
    p = 1.0 + s2 * (0.3333333333 + s2 * (0.2 + s2 * (0.14285714 + s2 * 0.11111111)))
    return e.astype(jnp.float32) * LN2 + 2.0 * s * p


def _post_body(scat_hbm, base_hbm, qz_out, vq_out,
               buf0, buf1, bb0, bb1, vqb, qzb, sem):
    c = lax.axis_index("c")
    s = lax.axis_index("s")
    wid = s * NC + c
    sl = wid // 2
    odd = wid % 2
    r0 = jnp.minimum(sl * (2 * ROWS) + odd * ROWS, I_T - ROWS)
    r0 = pl.multiple_of(r0, 8)
    loc = pl.multiple_of(r0 - sl * (2 * ROWS), 16)

    cps = [
        pltpu.async_copy(scat_hbm.at[0, sl], buf0, sem),
        pltpu.async_copy(scat_hbm.at[1, sl], buf1, sem),
        pltpu.async_copy(base_hbm.at[0, sl], bb0, sem),
        pltpu.async_copy(base_hbm.at[1, sl], bb1, sem),
    ]
    for cp in cps:
        cp.wait()

    def group(g, carry):
        rows = loc + g * 16 + lax.iota(jnp.int32, 16)
        rb = rows * K_C
        m = jnp.full((16,), -jnp.inf, jnp.float32)
        for k in range(K_C):
            v = plsc.load_gather(buf0, [rb + k]) + plsc.load_gather(buf1, [rb + k])
            plsc.store_scatter(buf0, [rb + k], v)
            m = jnp.maximum(m, v)
        z = jnp.zeros((16,), jnp.float32)
        for k in range(K_C):
            e = jnp.exp(plsc.load_gather(buf0, [rb + k]) - m)
            plsc.store_scatter(buf0, [rb + k], e)
            z = z + e
        r = 1.0 / z
        for k in range(K_C):
            q = plsc.load_gather(buf0, [rb + k]) * r
            qrows = g * 16 + lax.iota(jnp.int32, 16)
            plsc.store_scatter(qzb, [qrows, jnp.full((16,), k, jnp.int32)], q)
        base = bb0[pl.ds(loc + g * 16, 16)] + bb1[pl.ds(loc + g * 16, 16)]
        vqb[pl.ds(g * 16, 16)] = base + m + _log_f32(z)
        return carry

    lax.fori_loop(0, ROWS // 16, group, 0)

    pltpu.sync_copy(qzb, qz_out.at[pl.ds(r0, ROWS), :])
    pltpu.sync_copy(vqb, vq_out.at[pl.ds(r0, ROWS)])


_post_kernel = functools.partial(
    pl.kernel,
    mesh=plsc.VectorSubcoreMesh(core_axis_name="c", subcore_axis_name="s"),
    compiler_params=pltpu.CompilerParams(needs_layout_passes=False),
    out_type=[
        jax.ShapeDtypeStruct((I_T, K_C), jnp.float32),
        jax.ShapeDtypeStruct((I_T,), jnp.float32),
    ],
    scratch_types=[
        pltpu.VMEM((SCAT_SL,), jnp.float32),
        pltpu.VMEM((SCAT_SL,), jnp.float32),
        pltpu.VMEM((BASE_SL,), jnp.float32),
        pltpu.VMEM((BASE_SL,), jnp.float32),
        pltpu.VMEM((ROWS,), jnp.float32),
        pltpu.VMEM((ROWS, K_C), jnp.float32),
        pltpu.SemaphoreType.DMA,
    ],
)(_post_body)


def kernel(ii, jj, y, snr_logit):
    ii = ii.astype(jnp.int32)
    jj = jj.astype(jnp.int32)
    y = y.astype(jnp.int32)

    a_p, d_p = pl.pallas_call(
        _ad_body,
        out_shape=[jax.ShapeDtypeStruct((J_W,), jnp.float32)] * 2,
    )(snr_logit)

    zer = jnp.zeros((SCAT_SL,), jnp.float32)
    scat_p, base_p = _seg_kernel(ii, jj, y, a_p, d_p, zer)
    qz, vq = _post_kernel(scat_p, base_p)
    return qz, vq


# speedup vs baseline: 2.6803x; 1.5767x over previous
import functools

import jax
import jax.numpy as jnp
from jax import lax
from jax.experimental import pallas as pl
from jax.experimental.pallas import tpu as pltpu
from jax.experimental.pallas import tpu_sc as plsc

I_T = 10000
J_W = 1000
K_C = 32
N_L = 10000

NC = 2
NS = 16
NW = NC * NS

P_LBL = 320
CH = 64
I_PAD = 10240
SCAT_SL = I_PAD * K_C // NS
BASE_SL = I_PAD // NS
ROWS = 320
LN2 = 0.6931471805599453


def _ad_body(s_ref, a_ref, d_ref):
    s = s_ref[...]
    sig = jax.nn.sigmoid(s)
    noi = jax.nn.sigmoid(-s)
    tot = sig + noi
    a = jnp.log(noi / (K_C * tot))
    b = jnp.log((sig + noi / K_C) / tot)
    a_ref[...] = a
    d_ref[...] = b - a


def _seg_body(ii_hbm, jj_hbm, y_hbm, a_hbm, d_hbm, zer_hbm,
              scat_out, base_out,
              ii_v, jj_v, y_v, a_v, d_v, idx_b, val_b, iib_b, av_b,
              scat_sh, base_sh, sem):
    c = lax.axis_index("c")
    s = lax.axis_index("s")
    wid = s * NC + c
    start = wid * P_LBL
    l0 = jnp.minimum(start, N_L - P_LBL)
    l0 = pl.multiple_of(l0, 8)

    cps = [
        pltpu.async_copy(ii_hbm.at[pl.ds(l0, P_LBL)], ii_v, sem),
        pltpu.async_copy(jj_hbm.at[pl.ds(l0, P_LBL)], jj_v, sem),
        pltpu.async_copy(y_hbm.at[pl.ds(l0, P_LBL)], y_v, sem),
        pltpu.async_copy(a_hbm, a_v, sem),
        pltpu.async_copy(d_hbm, d_v, sem),
        pltpu.async_copy(zer_hbm, scat_sh.at[pl.ds(s * SCAT_SL, SCAT_SL)], sem),
        pltpu.async_copy(zer_hbm.at[pl.ds(0, BASE_SL)],
                         base_sh.at[pl.ds(s * BASE_SL, BASE_SL)], sem),
    ]
    for cp in cps:
        cp.wait()
    plsc.subcore_barrier()

    for chunk in range(P_LBL // CH):
        for v in range(CH // 16):
            off = chunk * CH + v * 16
            iiv = ii_v[pl.ds(off, 16)]
            jjv = jj_v[pl.ds(off, 16)]
            yv = y_v[pl.ds(off, 16)]
            av = plsc.load_gather(a_v, [jjv])
            dv = plsc.load_gather(d_v, [jjv])
            g = l0 + off + lax.iota(jnp.int32, 16)
            ok = g >= start
            zero = jnp.zeros((16,), jnp.float32)
            idx_b[pl.ds(v * 16, 16)] = iiv * K_C + yv
            val_b[pl.ds(v * 16, 16)] = jnp.where(ok, dv, zero)
            iib_b[pl.ds(v * 16, 16)] = iiv
            av_b[pl.ds(v * 16, 16)] = jnp.where(ok, av, zero)
        pltpu.sync_copy(val_b, scat_sh.at[idx_b], add=True)
        pltpu.sync_copy(av_b, base_sh.at[iib_b], add=True)

    plsc.subcore_barrier()
    pltpu.sync_copy(scat_sh.at[pl.ds(s * SCAT_SL, SCAT_SL)], scat_out.at[c, s])
    pltpu.sync_copy(base_sh.at[pl.ds(s * BASE_SL, BASE_SL)], base_out.at[c, s])


_seg_kernel = functools.partial(
    pl.kernel,
    mesh=plsc.VectorSubcoreMesh(core_axis_name="c", subcore_axis_name="s"),
    compiler_params=pltpu.CompilerParams(needs_layout_passes=False),
    out_type=[
        jax.ShapeDtypeStruct((NC, NS, SCAT_SL), jnp.float32),
        jax.ShapeDtypeStruct((NC, NS, BASE_SL), jnp.float32),
    ],
    scratch_types=[
        pltpu.VMEM((P_LBL,), jnp.int32),
        pltpu.VMEM((P_LBL,), jnp.int32),
        pltpu.VMEM((P_LBL,), jnp.int32),
        pltpu.VMEM((J_W,), jnp.float32),
        pltpu.VMEM((J_W,), jnp.float32),
        pltpu.VMEM((CH,), jnp.int32),
        pltpu.VMEM((CH,), jnp.float32),
        pltpu.VMEM((CH,), jnp.int32),
        pltpu.VMEM((CH,), jnp.float32),
        pltpu.VMEM_SHARED((I_PAD * K_C,), jnp.float32),
        pltpu.VMEM_SHARED((I_PAD,), jnp.float32),
        pltpu.SemaphoreType.DMA,
    ],
)(_seg_body)


def _log_f32(x):
    bits = plsc.bitcast(x, jnp.int32)
    e = (bits >> 23) - 127
    m = plsc.bitcast((bits & 0x7FFFFF) | 0x3F800000, jnp.float32)
    s = (m - 1.0) / (m + 1.0)
    s2 = s * s
    p = 1.0 + s2 * (0.3333333333 + s2 * (0.2 + s2 * (0.14285714 + s2 * 0.11111111)))
    return e.astype(jnp.float32) * LN2 + 2.0 * s * p


def _post_body(scat_hbm, base_hbm, qz_out, vq_out,
               buf0, buf1, bb0, bb1, vqb, qzb, sem):
    c = lax.axis_index("c")
    s = lax.axis_index("s")
    wid = s * NC + c
    sl = wid // 2
    odd = wid % 2
    r0 = jnp.minimum(sl * (2 * ROWS) + odd * ROWS, I_T - ROWS)
    r0 = pl.multiple_of(r0, 8)
    loc = pl.multiple_of(r0 - sl * (2 * ROWS), 16)

    cps = [
        pltpu.async_copy(scat_hbm.at[0, sl], buf0, sem),
        pltpu.async_copy(scat_hbm.at[1, sl], buf1, sem),
        pltpu.async_copy(base_hbm.at[0, sl], bb0, sem),
        pltpu.async_copy(base_hbm.at[1, sl], bb1, sem),
    ]
    for cp in cps:
        cp.wait()

    def _tree(xs, op):
        while len(xs) > 1:
            xs = [op(xs[i], xs[i + 1]) for i in range(0, len(xs) - 1, 2)] + (
                [xs[-1]] if len(xs) % 2 else [])
        return xs[0]

    def group(g, carry):
        rows = loc + g * 16 + lax.iota(jnp.int32, 16)
        rb = rows * K_C
        vs = [plsc.load_gather(buf0, [rb + k]) + plsc.load_gather(buf1, [rb + k])
              for k in range(K_C)]
        m = _tree(vs, jnp.maximum)
        es = [jnp.exp(v - m) for v in vs]
        z = _tree(es, lambda a, b: a + b)
        r = 1.0 / z
        qrows = g * 16 + lax.iota(jnp.int32, 16)
        for k in range(K_C):
            plsc.store_scatter(qzb, [qrows, jnp.full((16,), k, jnp.int32)],
                               es[k] * r)
        base = bb0[pl.ds(loc + g * 16, 16)] + bb1[pl.ds(loc + g * 16, 16)]
        vqb[pl.ds(g * 16, 16)] = base + m + _log_f32(z)
        return carry

    lax.fori_loop(0, ROWS // 16, group, 0)

    pltpu.sync_copy(qzb, qz_out.at[pl.ds(r0, ROWS), :])
    pltpu.sync_copy(vqb, vq_out.at[pl.ds(r0, ROWS)])


_post_kernel = functools.partial(
    pl.kernel,
    mesh=plsc.VectorSubcoreMesh(core_axis_name="c", subcore_axis_name="s"),
    compiler_params=pltpu.CompilerParams(needs_layout_passes=False),
    out_type=[
        jax.ShapeDtypeStruct((I_T, K_C), jnp.float32),
        jax.ShapeDtypeStruct((I_T,), jnp.float32),
    ],
    scratch_types=[
        pltpu.VMEM((SCAT_SL,), jnp.float32),
        pltpu.VMEM((SCAT_SL,), jnp.float32),
        pltpu.VMEM((BASE_SL,), jnp.float32),
        pltpu.VMEM((BASE_SL,), jnp.float32),
        pltpu.VMEM((ROWS,), jnp.float32),
        pltpu.VMEM((ROWS, K_C), jnp.float32),
        pltpu.SemaphoreType.DMA,
    ],
)(_post_body)


def kernel(ii, jj, y, snr_logit):
    ii = ii.astype(jnp.int32)
    jj = jj.astype(jnp.int32)
    y = y.astype(jnp.int32)

    a_p, d_p = pl.pallas_call(
        _ad_body,
        out_shape=[jax.ShapeDtypeStruct((J_W,), jnp.float32)] * 2,
    )(snr_logit)

    zer = jnp.zeros((SCAT_SL,), jnp.float32)
    scat_p, base_p = _seg_kernel(ii, jj, y, a_p, d_p, zer)
    qz, vq = _post_kernel(scat_p, base_p)
    return qz, vq
